# baseline (speedup 1.0000x reference)
import jax
import jax.numpy as jnp
from jax.experimental import pallas as pl


def _moe_kernel(x_ref, wg_ref, bg_ref, w1_ref, b1_ref, w2_ref, b2_ref, o_ref):
    xb = x_ref[...]
    g = jnp.dot(xb, wg_ref[...], preferred_element_type=jnp.float32)
    g = g + bg_ref[...]
    m = jnp.max(g, axis=-1, keepdims=True)
    p = 1.0 / jnp.sum(jnp.exp(g - m), axis=-1, keepdims=True)
    h = jnp.dot(xb, w1_ref[...], preferred_element_type=jnp.float32)
    h = h + b1_ref[...]
    h = 0.5 * h * (1.0 + jax.lax.erf(h * 0.7071067811865476))
    out = jnp.dot(h.astype(jnp.bfloat16), w2_ref[...],
                  preferred_element_type=jnp.float32)
    out = out + jnp.sum(b2_ref[...], axis=0, keepdims=True)
    o_ref[...] = out * p


def kernel(x, Wg, bg, W1, b1, W2, b2):
    B, S, D = x.shape
    E, _, H = W1.shape
    EH = E * H
    M = B * S
    TM = 512

    xf = x.reshape(M, D).astype(jnp.bfloat16)
    W1c = W1.transpose(1, 0, 2).reshape(D, EH).astype(jnp.bfloat16)
    b1c = b1.reshape(1, EH)
    W2c = W2.reshape(EH, D).astype(jnp.bfloat16)
    bg2 = bg.reshape(1, E)
    Wg16 = Wg.astype(jnp.bfloat16)

    out = pl.pallas_call(
        _moe_kernel,
        grid=(M // TM,),
        in_specs=[
            pl.BlockSpec((TM, D), lambda i: (i, 0)),
            pl.BlockSpec((D, E), lambda i: (0, 0)),
            pl.BlockSpec((1, E), lambda i: (0, 0)),
            pl.BlockSpec((D, EH), lambda i: (0, 0)),
            pl.BlockSpec((1, EH), lambda i: (0, 0)),
            pl.BlockSpec((EH, D), lambda i: (0, 0)),
            pl.BlockSpec((E, D), lambda i: (0, 0)),
        ],
        out_specs=pl.BlockSpec((TM, D), lambda i: (i, 0)),
        out_shape=jax.ShapeDtypeStruct((M, D), jnp.float32),
    )(xf, Wg16, bg2, W1c, b1c, W2c, b2)
    return out.reshape(B, S, D)


# speedup vs baseline: 1.1935x; 1.1935x over previous
import jax
import jax.numpy as jnp
from jax.experimental import pallas as pl


def _moe_kernel(x_ref, wg_ref, bg_ref, w1_ref, b1_ref, w2_ref, b2_ref, o_ref):
    xb = x_ref[...]
    g = jnp.dot(xb, wg_ref[...], preferred_element_type=jnp.float32)
    g = g + bg_ref[...]
    m = jnp.max(g, axis=-1, keepdims=True)
    p = 1.0 / jnp.sum(jnp.exp(g - m), axis=-1, keepdims=True)
    h = jnp.dot(xb.astype(jnp.bfloat16), w1_ref[...],
                preferred_element_type=jnp.float32)
    h = h + b1_ref[...]
    h = 0.5 * h * (1.0 + jax.lax.erf(h * 0.7071067811865476))
    out = jnp.dot(h.astype(jnp.bfloat16), w2_ref[...],
                  preferred_element_type=jnp.float32)
    out = out + jnp.sum(b2_ref[...], axis=0, keepdims=True)
    o_ref[...] = out * p


def kernel(x, Wg, bg, W1, b1, W2, b2):
    B, S, D = x.shape
    E, _, H = W1.shape
    EH = E * H
    M = B * S
    TM = 1024

    xf = x.reshape(M, D)
    W1c = W1.transpose(1, 0, 2).reshape(D, EH).astype(jnp.bfloat16)
    b1c = b1.reshape(1, EH)
    W2c = W2.reshape(EH, D).astype(jnp.bfloat16)
    bg2 = bg.reshape(1, E)

    out = pl.pallas_call(
        _moe_kernel,
        grid=(M // TM,),
        in_specs=[
            pl.BlockSpec((TM, D), lambda i: (i, 0)),
            pl.BlockSpec((D, E), lambda i: (0, 0)),
            pl.BlockSpec((1, E), lambda i: (0, 0)),
            pl.BlockSpec((D, EH), lambda i: (0, 0)),
            pl.BlockSpec((1, EH), lambda i: (0, 0)),
            pl.BlockSpec((EH, D), lambda i: (0, 0)),
            pl.BlockSpec((E, D), lambda i: (0, 0)),
        ],
        out_specs=pl.BlockSpec((TM, D), lambda i: (i, 0)),
        out_shape=jax.ShapeDtypeStruct((M, D), jnp.float32),
    )(xf, Wg, bg2, W1c, b1c, W2c, b2)
    return out.reshape(B, S, D)
